# baseline (speedup 1.0000x reference)
import functools

import jax
import jax.numpy as jnp
from jax import lax
from jax.experimental import pallas as pl
from jax.experimental.pallas import tpu as pltpu
from jax.experimental.pallas import tpu_sc as plsc

B = 8
MAX_SEQLEN = 2048
TOTAL_TOK = 8192
D = 1024

SC_B = 4
NC = 2
NS = 16
NW = NC * NS
SC_ROWS = SC_B * MAX_SEQLEN
RPW = SC_ROWS // NW
CHUNK = 32
NCHUNK = RPW // CHUNK
NB = 3
ZROWS = 16
LANES = 16
QPB = MAX_SEQLEN // RPW


def _sc_body(flat_hbm, cu_hbm, out_hbm, cu_v, zbuf, buf0, buf1, buf2,
             idx_v, idxb_v, sg0, sg1, sg2, ss0, ss1, ss2, sz):
    c = lax.axis_index("c")
    s = lax.axis_index("s")
    b = s // (NS // SC_B)
    t = s % (NS // SC_B)
    q = 2 * t + ((b + c) % 2)
    j0 = q * RPW
    row0 = b * MAX_SEQLEN + j0

    pltpu.sync_copy(cu_hbm, cu_v.at[pl.ds(0, B + 1)])

    def zfill(r, carry):
        for i in range(D // LANES):
            zbuf[r, pl.ds(i * LANES, LANES)] = jnp.zeros((LANES,), jnp.float32)
        return carry

    lax.fori_loop(0, ZROWS, zfill, 0)

    cu_vec = cu_v[...]
    iota = lax.broadcasted_iota(jnp.int32, (LANES,), 0)
    start = jnp.sum(jnp.where(iota == b, cu_vec, 0))
    end = jnp.sum(jnp.where(iota == b + 1, cu_vec, 0))
    seg_len = jnp.minimum(end - start, MAX_SEQLEN)
    nvalid = jnp.clip(seg_len - j0, 0, RPW)
    kfull = nvalid // CHUNK
    rem = nvalid % CHUNK
    kzero = kfull + (rem > 0).astype(jnp.int32)

    base = start + j0
    bufs = (buf0, buf1, buf2)
    sgs = (sg0, sg1, sg2)
    sss = (ss0, ss1, ss2)

    def out_half(k, h):
        return out_hbm.at[
            pl.ds(pl.multiple_of(row0 + k * CHUNK + h, ZROWS), ZROWS)]

    def out_chunk(k):
        return out_hbm.at[pl.ds(pl.multiple_of(row0 + k * CHUNK, CHUNK), CHUNK)]

    for k in range(NCHUNK):
        @pl.when(k >= kzero)
        def _():
            pltpu.async_copy(zbuf, out_half(k, 0), sz)
            pltpu.async_copy(zbuf, out_half(k, ZROWS), sz)

    for k in range(NCHUNK):
        p = k % NB
        if k >= NB:
            @pl.when(k - NB < kfull)
            def _():
                pltpu.make_async_copy(
                    bufs[p], out_chunk(k - NB), sss[p]).wait()

        for h in range(0, CHUNK, LANES):
            idx_v[k, pl.ds(h, LANES)] = jnp.minimum(
                base + (k * CHUNK + h) + iota, TOTAL_TOK - 1)

        @pl.when(k < kfull)
        def _():
            pltpu.async_copy(flat_hbm.at[idx_v.at[k]], bufs[p], sgs[p])

        if k >= 1:
            q1 = (k - 1) % NB

            @pl.when(k - 1 < kfull)
            def _():
                pltpu.make_async_copy(
                    flat_hbm.at[idx_v.at[k - 1]], bufs[q1], sgs[q1]).wait()
                pltpu.async_copy(bufs[q1], out_chunk(k - 1), sss[q1])

    @pl.when(NCHUNK - 1 < kfull)
    def _():
        q1 = (NCHUNK - 1) % NB
        pltpu.make_async_copy(
            flat_hbm.at[idx_v.at[NCHUNK - 1]], bufs[q1], sgs[q1]).wait()
        pltpu.async_copy(bufs[q1], out_chunk(NCHUNK - 1), sss[q1])

    for k in range(NCHUNK - NB, NCHUNK):
        @pl.when(k < kfull)
        def _():
            pltpu.make_async_copy(
                bufs[k % NB], out_chunk(k), sss[k % NB]).wait()

    bbase = base + kfull * CHUNK
    for h in range(0, CHUNK, LANES):
        idxb_v[pl.ds(h, LANES)] = jnp.minimum(bbase + h + iota, TOTAL_TOK - 1)

    @pl.when(rem > 0)
    def _():
        pltpu.async_copy(flat_hbm.at[idxb_v], buf0, sg0)
        pltpu.make_async_copy(flat_hbm.at[idxb_v], buf0, sg0).wait()

        def zero_row(r, carry):
            for i in range(D // LANES):
                buf0[r, pl.ds(i * LANES, LANES)] = jnp.zeros(
                    (LANES,), jnp.float32)
            return carry

        lax.fori_loop(rem, CHUNK, zero_row, 0)

    bchunk_dst = out_hbm.at[
        pl.ds(pl.multiple_of(row0 + kfull * CHUNK, CHUNK), CHUNK)]

    @pl.when(rem > 0)
    def _():
        pltpu.async_copy(buf0, bchunk_dst, sz)

    for k in range(NCHUNK):
        @pl.when(k >= kzero)
        def _():
            pltpu.make_async_copy(zbuf, out_half(k, 0), sz).wait()
            pltpu.make_async_copy(zbuf, out_half(k, ZROWS), sz).wait()

    @pl.when(rem > 0)
    def _():
        pltpu.make_async_copy(buf0, bchunk_dst, sz).wait()


def _sc_part(flat, cu_seqlens):
    mesh = plsc.VectorSubcoreMesh(core_axis_name="c", subcore_axis_name="s")
    out = pl.kernel(
        _sc_body,
        out_type=jax.ShapeDtypeStruct((SC_ROWS, D), jnp.float32),
        mesh=mesh,
        scratch_types=[
            pltpu.VMEM((LANES,), jnp.int32),
            pltpu.VMEM((ZROWS, D), jnp.float32),
            pltpu.VMEM((CHUNK, D), jnp.float32),
            pltpu.VMEM((CHUNK, D), jnp.float32),
            pltpu.VMEM((CHUNK, D), jnp.float32),
            pltpu.VMEM((NCHUNK, CHUNK), jnp.int32),
            pltpu.VMEM((CHUNK,), jnp.int32),
            pltpu.SemaphoreType.DMA,
            pltpu.SemaphoreType.DMA,
            pltpu.SemaphoreType.DMA,
            pltpu.SemaphoreType.DMA,
            pltpu.SemaphoreType.DMA,
            pltpu.SemaphoreType.DMA,
            pltpu.SemaphoreType.DMA,
        ],
        compiler_params=pltpu.CompilerParams(needs_layout_passes=False),
    )(flat, cu_seqlens)
    return out.reshape(SC_B, MAX_SEQLEN, D)


TC_B = B - SC_B
BS = 256
NBLK = MAX_SEQLEN // BS
NSTEP = TC_B * NBLK


def _tc_body(cu_ref, flat_hbm, out_ref, vbuf, sem):
    bi = pl.program_id(0)
    ji = pl.program_id(1)
    step = bi * NBLK + ji

    def window(stp):
        bb = stp // NBLK + SC_B
        jb = (stp % NBLK) * BS
        st = cu_ref[bb]
        ln = jnp.minimum(cu_ref[bb + 1] - st, MAX_SEQLEN)
        src = st + jb
        src_c = pl.multiple_of(
            jnp.minimum((src // 8) * 8, TOTAL_TOK - BS - 8), 8)
        return src_c, src - src_c, jb, ln

    def start_copy(stp, par):
        src_c, _, jb, ln = window(stp)

        @pl.when(jb < ln)
        def _():
            pltpu.make_async_copy(
                flat_hbm.at[pl.ds(src_c, BS + 8)],
                vbuf.at[par, pl.ds(0, BS + 8)], sem.at[par]).start()

    def wait_copy(stp, par):
        src_c, _, jb, ln = window(stp)

        @pl.when(jb < ln)
        def _():
            pltpu.make_async_copy(
                flat_hbm.at[pl.ds(src_c, BS + 8)],
                vbuf.at[par, pl.ds(0, BS + 8)], sem.at[par]).wait()

    par = lax.rem(step, 2)

    @pl.when(step == 0)
    def _():
        start_copy(step, par)

    @pl.when(step + 1 < NSTEP)
    def _():
        start_copy(step + 1, 1 - par)

    wait_copy(step, par)

    _, dd, jb, ln = window(step)
    dd8 = pl.multiple_of((dd // 8) * 8, 8)
    ddr = dd % 8
    rows_a = vbuf[par, pl.ds(dd8, BS), :]
    rows_b = vbuf[par, pl.ds(pl.multiple_of(dd8 + 8, 8), BS), :]
    rolled_a = pltpu.roll(rows_a, BS - ddr, 0)
    rolled_b = pltpu.roll(rows_b, 8 - ddr, 0)
    ri = lax.broadcasted_iota(jnp.int32, (BS, 1), 0)
    rows = jnp.where(ri < BS - ddr, rolled_a, rolled_b)
    mask = ri + jb < ln
    out_ref[0] = jnp.where(mask, rows, jnp.float32(0.0))


def _tc_part(flat, cu_seqlens):
    return pl.pallas_call(
        _tc_body,
        grid=(TC_B, NBLK),
        in_specs=[
            pl.BlockSpec(memory_space=pltpu.SMEM),
            pl.BlockSpec(memory_space=pl.ANY),
        ],
        out_specs=pl.BlockSpec((1, BS, D), lambda b, j: (b, j, 0)),
        out_shape=jax.ShapeDtypeStruct((TC_B, MAX_SEQLEN, D), jnp.float32),
        scratch_shapes=[
            pltpu.VMEM((2, 2 * BS + 16, D), jnp.float32),
            pltpu.SemaphoreType.DMA((2,)),
        ],
    )(cu_seqlens, flat)


@jax.jit
def kernel(flat, cu_seqlens):
    sc = _sc_part(flat, cu_seqlens)
    tc = _tc_part(flat, cu_seqlens)
    return jnp.concatenate([sc, tc], axis=0)


# speedup vs baseline: 1.7683x; 1.7683x over previous
import jax
import jax.numpy as jnp
from jax import lax
from jax.experimental import pallas as pl
from jax.experimental.pallas import tpu as pltpu
from jax.experimental.pallas import tpu_sc as plsc

B = 8
MAX_SEQLEN = 2048
TOTAL_TOK = 8192
D = 1024

NC = 2
NS = 16
NW = NC * NS
TOTAL_ROWS = B * MAX_SEQLEN
RPW = TOTAL_ROWS // NW
CHUNK = 32
NCHUNK = RPW // CHUNK
NB = 3
ZROWS = 16
LANES = 16


def _body(flat_hbm, cu_hbm, out_hbm, cu_v, zbuf, buf0, buf1, buf2,
          idx_v, idxb_v, sg0, sg1, sg2, ss0, ss1, ss2, sz):
    c = lax.axis_index("c")
    s = lax.axis_index("s")
    b = s // 2
    q = 2 * (s % 2) + ((b + c) % 2)
    j0 = q * RPW
    row0 = b * MAX_SEQLEN + j0

    pltpu.sync_copy(cu_hbm, cu_v.at[pl.ds(0, B + 1)])

    def zfill(r, carry):
        for i in range(D // LANES):
            zbuf[r, pl.ds(i * LANES, LANES)] = jnp.zeros((LANES,), jnp.float32)
        return carry

    lax.fori_loop(0, ZROWS, zfill, 0)

    cu_vec = cu_v[...]
    iota = lax.broadcasted_iota(jnp.int32, (LANES,), 0)
    start = jnp.sum(jnp.where(iota == b, cu_vec, 0))
    end = jnp.sum(jnp.where(iota == b + 1, cu_vec, 0))
    seg_len = jnp.minimum(end - start, MAX_SEQLEN)
    nvalid = jnp.clip(seg_len - j0, 0, RPW)
    kfull = nvalid // CHUNK
    rem = nvalid % CHUNK
    kzero = kfull + (rem > 0).astype(jnp.int32)

    base = start + j0
    bufs = (buf0, buf1, buf2)
    sgs = (sg0, sg1, sg2)
    sss = (ss0, ss1, ss2)

    def out_half(k, h):
        return out_hbm.at[
            pl.ds(pl.multiple_of(row0 + k * CHUNK + h, ZROWS), ZROWS)]

    def out_chunk(k):
        return out_hbm.at[pl.ds(pl.multiple_of(row0 + k * CHUNK, CHUNK), CHUNK)]

    for k in range(NCHUNK):
        @pl.when(k >= kzero)
        def _():
            pltpu.async_copy(zbuf, out_half(k, 0), sz)
            pltpu.async_copy(zbuf, out_half(k, ZROWS), sz)

    for k in range(NCHUNK):
        p = k % NB
        if k >= NB:
            @pl.when(k - NB < kfull)
            def _():
                pltpu.make_async_copy(
                    bufs[p], out_chunk(k - NB), sss[p]).wait()

        for h in range(0, CHUNK, LANES):
            idx_v[k, pl.ds(h, LANES)] = jnp.minimum(
                base + (k * CHUNK + h) + iota, TOTAL_TOK - 1)

        @pl.when(k < kfull)
        def _():
            pltpu.async_copy(flat_hbm.at[idx_v.at[k]], bufs[p], sgs[p])

        if k >= 1:
            q1 = (k - 1) % NB

            @pl.when(k - 1 < kfull)
            def _():
                pltpu.make_async_copy(
                    flat_hbm.at[idx_v.at[k - 1]], bufs[q1], sgs[q1]).wait()
                pltpu.async_copy(bufs[q1], out_chunk(k - 1), sss[q1])

    @pl.when(NCHUNK - 1 < kfull)
    def _():
        q1 = (NCHUNK - 1) % NB
        pltpu.make_async_copy(
            flat_hbm.at[idx_v.at[NCHUNK - 1]], bufs[q1], sgs[q1]).wait()
        pltpu.async_copy(bufs[q1], out_chunk(NCHUNK - 1), sss[q1])

    for k in range(NCHUNK - NB, NCHUNK):
        @pl.when(k < kfull)
        def _():
            pltpu.make_async_copy(
                bufs[k % NB], out_chunk(k), sss[k % NB]).wait()

    bbase = base + kfull * CHUNK
    for h in range(0, CHUNK, LANES):
        idxb_v[pl.ds(h, LANES)] = jnp.minimum(bbase + h + iota, TOTAL_TOK - 1)

    @pl.when(rem > 0)
    def _():
        pltpu.async_copy(flat_hbm.at[idxb_v], buf0, sg0)
        pltpu.make_async_copy(flat_hbm.at[idxb_v], buf0, sg0).wait()

        def zero_row(r, carry):
            for i in range(D // LANES):
                buf0[r, pl.ds(i * LANES, LANES)] = jnp.zeros(
                    (LANES,), jnp.float32)
            return carry

        lax.fori_loop(rem, CHUNK, zero_row, 0)

    bchunk_dst = out_hbm.at[
        pl.ds(pl.multiple_of(row0 + kfull * CHUNK, CHUNK), CHUNK)]

    @pl.when(rem > 0)
    def _():
        pltpu.async_copy(buf0, bchunk_dst, sz)

    for k in range(NCHUNK):
        @pl.when(k >= kzero)
        def _():
            pltpu.make_async_copy(zbuf, out_half(k, 0), sz).wait()
            pltpu.make_async_copy(zbuf, out_half(k, ZROWS), sz).wait()

    @pl.when(rem > 0)
    def _():
        pltpu.make_async_copy(buf0, bchunk_dst, sz).wait()


@jax.jit
def kernel(flat, cu_seqlens):
    mesh = plsc.VectorSubcoreMesh(core_axis_name="c", subcore_axis_name="s")
    out = pl.kernel(
        _body,
        out_type=jax.ShapeDtypeStruct((TOTAL_ROWS, D), jnp.float32),
        mesh=mesh,
        scratch_types=[
            pltpu.VMEM((LANES,), jnp.int32),
            pltpu.VMEM((ZROWS, D), jnp.float32),
            pltpu.VMEM((CHUNK, D), jnp.float32),
            pltpu.VMEM((CHUNK, D), jnp.float32),
            pltpu.VMEM((CHUNK, D), jnp.float32),
            pltpu.VMEM((NCHUNK, CHUNK), jnp.int32),
            pltpu.VMEM((CHUNK,), jnp.int32),
            pltpu.SemaphoreType.DMA,
            pltpu.SemaphoreType.DMA,
            pltpu.SemaphoreType.DMA,
            pltpu.SemaphoreType.DMA,
            pltpu.SemaphoreType.DMA,
            pltpu.SemaphoreType.DMA,
            pltpu.SemaphoreType.DMA,
        ],
        compiler_params=pltpu.CompilerParams(needs_layout_passes=False),
    )(flat, cu_seqlens)
    return out.reshape(B, MAX_SEQLEN, D)
